# baseline (speedup 1.0000x reference)
import jax
import jax.numpy as jnp
from jax.experimental import pallas as pl
from jax.experimental.pallas import tpu as pltpu

N = 10000
R = 2
INDIM = 128
OUTDIM = 128

TM = 40
NT = N // TM
NBUF = 8


def _fused_kernel(x_ref, w_ref, a_hbm, o_ref, fw_ref, slots, sem):
    m = pl.program_id(0)

    def a_copy(block, slot):
        return pltpu.make_async_copy(
            a_hbm.at[pl.ds(block * TM, TM), :], slots.at[slot], sem.at[slot])

    @pl.when(m == 0)
    def _prologue():
        for i in range(NBUF):
            a_copy(i, i).start()
        for r in range(R):
            fw_ref[r * N:(r + 1) * N, :] = jnp.dot(
                x_ref[...], w_ref[r], preferred_element_type=jnp.float32)

    slot = jax.lax.rem(m, NBUF)
    a_copy(m, slot).wait()
    o_ref[...] = jnp.dot(slots[slot], fw_ref[...],
                         preferred_element_type=jnp.float32)

    @pl.when(m + NBUF < NT)
    def _refill():
        a_copy(m + NBUF, slot).start()


@jax.jit
def kernel(X, A, W_F):
    return pl.pallas_call(
        _fused_kernel,
        grid=(NT,),
        in_specs=[
            pl.BlockSpec((N, INDIM), lambda m: (0, 0)),
            pl.BlockSpec((R, INDIM, OUTDIM), lambda m: (0, 0, 0)),
            pl.BlockSpec(memory_space=pltpu.MemorySpace.HBM),
        ],
        out_specs=pl.BlockSpec((TM, OUTDIM), lambda m: (m, 0)),
        out_shape=jax.ShapeDtypeStruct((N, OUTDIM), jnp.float32),
        scratch_shapes=[
            pltpu.VMEM((R * N, OUTDIM), jnp.float32),
            pltpu.VMEM((NBUF, TM, R * N), jnp.float32),
            pltpu.SemaphoreType.DMA((NBUF,)),
        ],
        compiler_params=pltpu.CompilerParams(
            dimension_semantics=("arbitrary",),
        ),
    )(X, W_F, A)


# speedup vs baseline: 1.0321x; 1.0321x over previous
import jax
import jax.numpy as jnp
from jax.experimental import pallas as pl
from jax.experimental.pallas import tpu as pltpu

N = 10000
R = 2
INDIM = 128
OUTDIM = 128

TM = 80
NT = N // TM
NBUF = 6


def _fused_kernel(x_ref, w_ref, a_hbm, o_ref, fw_ref, slots, sem):
    m = pl.program_id(0)

    def a_copy(block, slot):
        return pltpu.make_async_copy(
            a_hbm.at[pl.ds(block * TM, TM), :], slots.at[slot], sem.at[slot])

    @pl.when(m == 0)
    def _prologue():
        for i in range(NBUF):
            a_copy(i, i).start()
        for r in range(R):
            fw_ref[r * N:(r + 1) * N, :] = jnp.dot(
                x_ref[...], w_ref[r], preferred_element_type=jnp.float32)

    slot = jax.lax.rem(m, NBUF)
    a_copy(m, slot).wait()
    o_ref[...] = jnp.dot(slots[slot], fw_ref[...],
                         preferred_element_type=jnp.float32)

    @pl.when(m + NBUF < NT)
    def _refill():
        a_copy(m + NBUF, slot).start()


@jax.jit
def kernel(X, A, W_F):
    return pl.pallas_call(
        _fused_kernel,
        grid=(NT,),
        in_specs=[
            pl.BlockSpec((N, INDIM), lambda m: (0, 0)),
            pl.BlockSpec((R, INDIM, OUTDIM), lambda m: (0, 0, 0)),
            pl.BlockSpec(memory_space=pltpu.MemorySpace.HBM),
        ],
        out_specs=pl.BlockSpec((TM, OUTDIM), lambda m: (m, 0)),
        out_shape=jax.ShapeDtypeStruct((N, OUTDIM), jnp.float32),
        scratch_shapes=[
            pltpu.VMEM((R * N, OUTDIM), jnp.float32),
            pltpu.VMEM((NBUF, TM, R * N), jnp.float32),
            pltpu.SemaphoreType.DMA((NBUF,)),
        ],
        compiler_params=pltpu.CompilerParams(
            dimension_semantics=("arbitrary",),
        ),
    )(X, W_F, A)
